# baseline (speedup 1.0000x reference)
import functools

import jax
import jax.numpy as jnp
from jax import lax
from jax.experimental import pallas as pl
from jax.experimental.pallas import tpu as pltpu
from jax.experimental.pallas import tpu_sc as plsc

EPS = 1e-06

_PERMS = (
    (2, 1, 0),
    (1, 2, 0),
    (0, 1, 2),
    (1, 0, 2),
    (2, 0, 1),
    (0, 1, 2),
    (0, 2, 1),
    (0, 1, 2),
)

_L = 16
_NC = 2
_NW = 32
_DQ = 33
_NV = 50
_TCB = 1024


def _round_to_bf16(x):
    u = lax.bitcast_convert_type(x, jnp.int32)
    lsb = lax.shift_right_logical(u, 16) & jnp.int32(1)
    r = (u + jnp.int32(0x7FFF) + lsb) & jnp.int32(-65536)
    return lax.bitcast_convert_type(r, jnp.float32)


def _tc_pid_body(q_ref, k_ref, ikn_ref, pid_ref):
    q = _round_to_bf16(q_ref[...])
    s = []
    for mm in range(3):
        km = k_ref[mm, :_DQ][None, :]
        s.append(jnp.sum(q * km, axis=1) * ikn_ref[mm, 0])
    zero = jnp.zeros(s[0].shape, jnp.int32)
    pid_ref[...] = (
        jnp.where(s[0] >= s[1], jnp.int32(4), zero)
        + jnp.where(s[0] >= s[2], jnp.int32(2), zero)
        + jnp.where(s[1] >= s[2], jnp.int32(1), zero)
    )


def _sc_body(pid_hbm, vsp_hbm, out_hbm, idxv, vspv, buf, qw):
    wid = lax.axis_index("s") * _NC + lax.axis_index("c")
    qbase = wid * qw

    pltpu.sync_copy(pid_hbm.at[pl.ds(qbase, qw)], idxv)
    pltpu.sync_copy(vsp_hbm, vspv)

    for p in range(9):
        r = p // 3

        def bloop(g, carry):
            pidv = idxv[pl.ds(g * _L, _L)]
            gsel = jnp.full((_L,), _PERMS[0][r], jnp.int32)
            for k in range(1, 8):
                gsel = jnp.where(pidv == k, jnp.int32(_PERMS[k][r]), gsel)
            c0 = gsel == 0
            c1 = gsel == 1
            for i in range(_NV):
                base = ((p * _NV + i) * 3) * _L
                val = jnp.where(
                    c0, vspv[pl.ds(base, _L)],
                    jnp.where(c1, vspv[pl.ds(base + _L, _L)],
                              vspv[pl.ds(base + 2 * _L, _L)]),
                )
                buf[i, pl.ds(g * _L, _L)] = val
            return carry

        lax.fori_loop(0, qw // _L, bloop, 0)
        pltpu.sync_copy(buf, out_hbm.at[p, :, pl.ds(qbase, qw)])


def kernel(queries, mem_keys, mem_values, top_num):
    del top_num
    bsz, dq = queries.shape
    m, nv, _ = mem_values.shape
    assert (m, dq, nv) == (3, _DQ, _NV)
    qw = bsz // _NW

    kpad = jnp.zeros((8, 128), jnp.float32)
    kpad = kpad.at[:m, :dq].set(_round_to_bf16(mem_keys))
    knorm = jnp.maximum(jnp.linalg.norm(mem_keys, axis=1), EPS)
    ikn = jnp.zeros((8, 128), jnp.float32)
    ikn = ikn.at[:m, 0].set((1.0 / knorm).astype(jnp.float32))
    vsp = jnp.transpose(mem_values, (2, 1, 0))
    vsp = jnp.tile(vsp, (3, 1, 1))
    vsp = jnp.broadcast_to(vsp[..., None], (9, nv, 3, _L)).astype(jnp.float32)
    vsp = vsp.reshape(-1)

    pid = pl.pallas_call(
        _tc_pid_body,
        grid=(bsz // _TCB,),
        in_specs=[
            pl.BlockSpec((_TCB, dq), lambda i: (i, 0)),
            pl.BlockSpec((8, 128), lambda i: (0, 0)),
            pl.BlockSpec((8, 128), lambda i: (0, 0)),
        ],
        out_specs=pl.BlockSpec((_TCB,), lambda i: (i,)),
        out_shape=jax.ShapeDtypeStruct((bsz,), jnp.int32),
    )(queries, kpad, ikn)

    mesh = plsc.VectorSubcoreMesh(core_axis_name="c", subcore_axis_name="s")
    run = pl.kernel(
        functools.partial(_sc_body, qw=qw),
        out_type=jax.ShapeDtypeStruct((9, nv, bsz), jnp.float32),
        mesh=mesh,
        scratch_types=[
            pltpu.VMEM((qw,), jnp.int32),
            pltpu.VMEM((9 * nv * 3 * _L,), jnp.float32),
            pltpu.VMEM((nv, qw), jnp.float32),
        ],
    )
    out9 = run(pid, vsp)
    out4 = out9.reshape(3, 3, nv, bsz)
    return jnp.transpose(out4, (3, 0, 2, 1))


# speedup vs baseline: 2.1052x; 2.1052x over previous
import functools

import jax
import jax.numpy as jnp
from jax import lax
from jax.experimental import pallas as pl
from jax.experimental.pallas import tpu as pltpu
from jax.experimental.pallas import tpu_sc as plsc

EPS = 1e-06

_PERMS = (
    (2, 1, 0),
    (1, 2, 0),
    (0, 1, 2),
    (1, 0, 2),
    (2, 0, 1),
    (0, 1, 2),
    (0, 2, 1),
    (0, 1, 2),
)

_L = 16
_NC = 2
_NW = 32
_DQ = 33
_NV = 50
_TCB = 1024


def _round_to_bf16(x):
    u = lax.bitcast_convert_type(x, jnp.int32)
    lsb = lax.shift_right_logical(u, 16) & jnp.int32(1)
    r = (u + jnp.int32(0x7FFF) + lsb) & jnp.int32(-65536)
    return lax.bitcast_convert_type(r, jnp.float32)


def _tc_pid_body(q_ref, k_ref, ikn_ref, pid_ref):
    q = _round_to_bf16(q_ref[...])
    s = []
    for mm in range(3):
        km = k_ref[mm, :_DQ][None, :]
        s.append(jnp.sum(q * km, axis=1) * ikn_ref[mm, 0])
    zero = jnp.zeros(s[0].shape, jnp.int32)
    pid_ref[...] = (
        jnp.where(s[0] >= s[1], jnp.int32(4), zero)
        + jnp.where(s[0] >= s[2], jnp.int32(2), zero)
        + jnp.where(s[1] >= s[2], jnp.int32(1), zero)
    )


def _sc_body(pid_hbm, vsp_hbm, out_hbm, idxv, vspv, buf, qw):
    wid = lax.axis_index("s") * _NC + lax.axis_index("c")
    qbase = wid * qw

    pltpu.sync_copy(pid_hbm.at[pl.ds(qbase, qw)], idxv)
    pltpu.sync_copy(vsp_hbm, vspv)

    for p in range(9):
        r = p // 3

        @functools.partial(plsc.parallel_loop, 0, qw // _L, unroll=2)
        def bloop(g, p=p, r=r):
            pidv = idxv[pl.ds(g * _L, _L)]
            gsel = jnp.full((_L,), _PERMS[0][r], jnp.int32)
            for k in range(1, 8):
                gsel = jnp.where(pidv == k, jnp.int32(_PERMS[k][r]), gsel)
            c0 = gsel == 0
            c1 = gsel == 1
            for i in range(_NV):
                base = ((p * _NV + i) * 3) * _L
                val = jnp.where(
                    c0, vspv[pl.ds(base, _L)],
                    jnp.where(c1, vspv[pl.ds(base + _L, _L)],
                              vspv[pl.ds(base + 2 * _L, _L)]),
                )
                buf[i, pl.ds(g * _L, _L)] = val

        pltpu.sync_copy(buf, out_hbm.at[p, :, pl.ds(qbase, qw)])


def kernel(queries, mem_keys, mem_values, top_num):
    del top_num
    bsz, dq = queries.shape
    m, nv, _ = mem_values.shape
    assert (m, dq, nv) == (3, _DQ, _NV)
    qw = bsz // _NW

    kpad = jnp.zeros((8, 128), jnp.float32)
    kpad = kpad.at[:m, :dq].set(_round_to_bf16(mem_keys))
    knorm = jnp.maximum(jnp.linalg.norm(mem_keys, axis=1), EPS)
    ikn = jnp.zeros((8, 128), jnp.float32)
    ikn = ikn.at[:m, 0].set((1.0 / knorm).astype(jnp.float32))
    vsp = jnp.transpose(mem_values, (2, 1, 0))
    vsp = jnp.tile(vsp, (3, 1, 1))
    vsp = jnp.broadcast_to(vsp[..., None], (9, nv, 3, _L)).astype(jnp.float32)
    vsp = vsp.reshape(-1)

    pid = pl.pallas_call(
        _tc_pid_body,
        grid=(bsz // _TCB,),
        in_specs=[
            pl.BlockSpec((_TCB, dq), lambda i: (i, 0)),
            pl.BlockSpec((8, 128), lambda i: (0, 0)),
            pl.BlockSpec((8, 128), lambda i: (0, 0)),
        ],
        out_specs=pl.BlockSpec((_TCB,), lambda i: (i,)),
        out_shape=jax.ShapeDtypeStruct((bsz,), jnp.int32),
    )(queries, kpad, ikn)

    mesh = plsc.VectorSubcoreMesh(core_axis_name="c", subcore_axis_name="s")
    run = pl.kernel(
        functools.partial(_sc_body, qw=qw),
        out_type=jax.ShapeDtypeStruct((9, nv, bsz), jnp.float32),
        mesh=mesh,
        scratch_types=[
            pltpu.VMEM((qw,), jnp.int32),
            pltpu.VMEM((9 * nv * 3 * _L,), jnp.float32),
            pltpu.VMEM((nv, qw), jnp.float32),
        ],
    )
    out9 = run(pid, vsp)
    out4 = out9.reshape(3, 3, nv, bsz)
    return jnp.transpose(out4, (3, 0, 2, 1))


# speedup vs baseline: 2.2817x; 1.0838x over previous
import functools

import jax
import jax.numpy as jnp
from jax import lax
from jax.experimental import pallas as pl
from jax.experimental.pallas import tpu as pltpu
from jax.experimental.pallas import tpu_sc as plsc

EPS = 1e-06

_PERMS = (
    (2, 1, 0),
    (1, 2, 0),
    (0, 1, 2),
    (1, 0, 2),
    (2, 0, 1),
    (0, 1, 2),
    (0, 2, 1),
    (0, 1, 2),
)

_L = 16
_NC = 2
_NW = 32
_DQ = 33
_NV = 50
_TCB = 1024


def _round_to_bf16(x):
    u = lax.bitcast_convert_type(x, jnp.int32)
    lsb = lax.shift_right_logical(u, 16) & jnp.int32(1)
    r = (u + jnp.int32(0x7FFF) + lsb) & jnp.int32(-65536)
    return lax.bitcast_convert_type(r, jnp.float32)


def _tc_pid_body(q_ref, k_ref, ikn_ref, pid_ref):
    q = _round_to_bf16(q_ref[...])
    s = []
    for mm in range(3):
        km = k_ref[mm, :_DQ][None, :]
        s.append(jnp.sum(q * km, axis=1) * ikn_ref[mm, 0])
    zero = jnp.zeros(s[0].shape, jnp.int32)
    pid_ref[...] = (
        jnp.where(s[0] >= s[1], jnp.int32(4), zero)
        + jnp.where(s[0] >= s[2], jnp.int32(2), zero)
        + jnp.where(s[1] >= s[2], jnp.int32(1), zero)
    )


def _sc_body(pid_hbm, vsp_hbm, out_hbm, idxv, vspv, buf0, buf1, sem, qw):
    wid = lax.axis_index("s") * _NC + lax.axis_index("c")
    qbase = wid * qw

    pltpu.sync_copy(pid_hbm.at[pl.ds(qbase, qw)], idxv)
    pltpu.sync_copy(vsp_hbm, vspv)

    bufs = (buf0, buf1)
    copies = [None, None]
    for p in range(9):
        r = p // 3
        buf = bufs[p % 2]
        if copies[p % 2] is not None:
            copies[p % 2].wait()

        @functools.partial(plsc.parallel_loop, 0, qw // _L, unroll=2)
        def bloop(g, p=p, r=r, buf=buf):
            pidv = idxv[pl.ds(g * _L, _L)]
            gsel = jnp.full((_L,), _PERMS[0][r], jnp.int32)
            for k in range(1, 8):
                gsel = jnp.where(pidv == k, jnp.int32(_PERMS[k][r]), gsel)
            c0 = gsel == 0
            c1 = gsel == 1
            for i in range(_NV):
                base = ((p * _NV + i) * 3) * _L
                val = jnp.where(
                    c0, vspv[pl.ds(base, _L)],
                    jnp.where(c1, vspv[pl.ds(base + _L, _L)],
                              vspv[pl.ds(base + 2 * _L, _L)]),
                )
                buf[i, pl.ds(g * _L, _L)] = val

        plsc.subcore_barrier()
        copies[p % 2] = pltpu.async_copy(
            buf, out_hbm.at[p, :, pl.ds(qbase, qw)], sem
        )
    copies[1].wait()
    copies[0].wait()


def kernel(queries, mem_keys, mem_values, top_num):
    del top_num
    bsz, dq = queries.shape
    m, nv, _ = mem_values.shape
    assert (m, dq, nv) == (3, _DQ, _NV)
    qw = bsz // _NW

    kpad = jnp.zeros((8, 128), jnp.float32)
    kpad = kpad.at[:m, :dq].set(_round_to_bf16(mem_keys))
    knorm = jnp.maximum(jnp.linalg.norm(mem_keys, axis=1), EPS)
    ikn = jnp.zeros((8, 128), jnp.float32)
    ikn = ikn.at[:m, 0].set((1.0 / knorm).astype(jnp.float32))
    vsp = jnp.transpose(mem_values, (2, 1, 0))
    vsp = jnp.tile(vsp, (3, 1, 1))
    vsp = jnp.broadcast_to(vsp[..., None], (9, nv, 3, _L)).astype(jnp.float32)
    vsp = vsp.reshape(-1)

    pid = pl.pallas_call(
        _tc_pid_body,
        grid=(bsz // _TCB,),
        in_specs=[
            pl.BlockSpec((_TCB, dq), lambda i: (i, 0)),
            pl.BlockSpec((8, 128), lambda i: (0, 0)),
            pl.BlockSpec((8, 128), lambda i: (0, 0)),
        ],
        out_specs=pl.BlockSpec((_TCB,), lambda i: (i,)),
        out_shape=jax.ShapeDtypeStruct((bsz,), jnp.int32),
    )(queries, kpad, ikn)

    mesh = plsc.VectorSubcoreMesh(core_axis_name="c", subcore_axis_name="s")
    run = pl.kernel(
        functools.partial(_sc_body, qw=qw),
        out_type=jax.ShapeDtypeStruct((9, nv, bsz), jnp.float32),
        mesh=mesh,
        scratch_types=[
            pltpu.VMEM((qw,), jnp.int32),
            pltpu.VMEM((9 * nv * 3 * _L,), jnp.float32),
            pltpu.VMEM((nv, qw), jnp.float32),
            pltpu.VMEM((nv, qw), jnp.float32),
            pltpu.SemaphoreType.DMA,
        ],
    )
    out9 = run(pid, vsp)
    out4 = out9.reshape(3, 3, nv, bsz)
    return jnp.transpose(out4, (3, 0, 2, 1))
